# Initial kernel scaffold; baseline (speedup 1.0000x reference)
import jax
import jax.numpy as jnp
from jax.experimental import pallas as pl


def kernel(x, W):
    raise NotImplementedError("write your pallas kernel here")


# speedup vs baseline: 2.2858x; 2.2858x over previous
import jax
import jax.numpy as jnp
from jax import lax
from jax.experimental import pallas as pl
from jax.experimental.pallas import tpu as pltpu
from jax.experimental.pallas import tpu_sc as plsc

NUM_EXPERTS = 8
D_MODEL = 2048
TOKENS = 32768
TOK_BLK = 2048
NUM_CORES = 2
NUM_SUBCORES = 16
LANES = 16
NW = NUM_CORES * NUM_SUBCORES
TPW = TOKENS // NW


def _gate_matmul_body(w_ref, x_ref, out_ref):
    out_ref[...] = lax.dot_general(
        w_ref[...], x_ref[...],
        dimension_numbers=(((1,), (1,)), ((), ())),
        preferred_element_type=jnp.float32,
    )


def _routing_body(logits_hbm, w_hbm, idx_hbm, lg_v, w_v, idx_v):
    wid = lax.axis_index("s") * NUM_CORES + lax.axis_index("c")
    base = wid * TPW
    pltpu.sync_copy(logits_hbm.at[:, pl.ds(base, TPW)], lg_v)

    def step(i, carry):
        off = pl.multiple_of(i * LANES, LANES)
        vs = [lg_v[e, pl.ds(off, LANES)] for e in range(NUM_EXPERTS)]
        m = vs[0]
        idx = jnp.zeros((LANES,), jnp.int32)
        for e in range(1, NUM_EXPERTS):
            gt = vs[e] > m
            m = jnp.where(gt, vs[e], m)
            idx = jnp.where(gt, jnp.full((LANES,), e, jnp.int32), idx)
        ssum = jnp.zeros((LANES,), jnp.float32)
        for e in range(NUM_EXPERTS):
            ssum = ssum + jnp.exp(vs[e] - m)
        w_v[pl.ds(off, LANES)] = 1.0 / ssum
        idx_v[pl.ds(off, LANES)] = idx
        return carry

    lax.fori_loop(0, TPW // LANES, step, 0)
    pltpu.sync_copy(w_v, w_hbm.at[pl.ds(base, TPW)])
    pltpu.sync_copy(idx_v, idx_hbm.at[pl.ds(base, TPW)])


def kernel(x, W):
    logits_t = pl.pallas_call(
        _gate_matmul_body,
        grid=(TOKENS // TOK_BLK,),
        in_specs=[
            pl.BlockSpec((NUM_EXPERTS, D_MODEL), lambda i: (0, 0)),
            pl.BlockSpec((TOK_BLK, D_MODEL), lambda i: (i, 0)),
        ],
        out_specs=pl.BlockSpec((NUM_EXPERTS, TOK_BLK), lambda i: (0, i)),
        out_shape=jax.ShapeDtypeStruct((NUM_EXPERTS, TOKENS), jnp.float32),
    )(W.astype(jnp.float32), x.astype(jnp.float32))

    weights, indices = pl.kernel(
        _routing_body,
        out_type=[
            jax.ShapeDtypeStruct((TOKENS,), jnp.float32),
            jax.ShapeDtypeStruct((TOKENS,), jnp.int32),
        ],
        mesh=plsc.VectorSubcoreMesh(core_axis_name="c", subcore_axis_name="s"),
        scratch_types=[
            pltpu.VMEM((NUM_EXPERTS, TPW), jnp.float32),
            pltpu.VMEM((TPW,), jnp.float32),
            pltpu.VMEM((TPW,), jnp.int32),
        ],
    )(logits_t)

    return (weights.reshape(TOKENS, 1).astype(x.dtype),
            indices.reshape(TOKENS, 1))


# speedup vs baseline: 2.3199x; 1.0149x over previous
import jax
import jax.numpy as jnp
from jax import lax
from jax.experimental import pallas as pl
from jax.experimental.pallas import tpu as pltpu
from jax.experimental.pallas import tpu_sc as plsc

NUM_EXPERTS = 8
D_MODEL = 2048
TOKENS = 32768
TOK_BLK = 1024
NUM_CORES = 2
NUM_SUBCORES = 16
LANES = 16
NW = NUM_CORES * NUM_SUBCORES
TPW = TOKENS // NW


def _gate_matmul_body(w_ref, x_ref, out_ref):
    out_ref[...] = lax.dot_general(
        w_ref[...], x_ref[...],
        dimension_numbers=(((1,), (1,)), ((), ())),
        preferred_element_type=jnp.float32,
    )


def _routing_body(logits_hbm, w_hbm, idx_hbm, lg_v, w_v, idx_v):
    wid = lax.axis_index("s") * NUM_CORES + lax.axis_index("c")
    base = wid * TPW
    pltpu.sync_copy(logits_hbm.at[:, pl.ds(base, TPW)], lg_v)

    def step(i, carry):
        off = pl.multiple_of(i * LANES, LANES)
        vs = [lg_v[e, pl.ds(off, LANES)] for e in range(NUM_EXPERTS)]
        m = vs[0]
        idx = jnp.zeros((LANES,), jnp.int32)
        for e in range(1, NUM_EXPERTS):
            gt = vs[e] > m
            m = jnp.where(gt, vs[e], m)
            idx = jnp.where(gt, jnp.full((LANES,), e, jnp.int32), idx)
        ssum = jnp.zeros((LANES,), jnp.float32)
        for e in range(NUM_EXPERTS):
            ssum = ssum + jnp.exp(vs[e] - m)
        w_v[pl.ds(off, LANES)] = 1.0 / ssum
        idx_v[pl.ds(off, LANES)] = idx
        return carry

    lax.fori_loop(0, TPW // LANES, step, 0)
    pltpu.sync_copy(w_v, w_hbm.at[pl.ds(base, TPW)])
    pltpu.sync_copy(idx_v, idx_hbm.at[pl.ds(base, TPW)])


def kernel(x, W):
    logits_t = pl.pallas_call(
        _gate_matmul_body,
        grid=(TOKENS // TOK_BLK,),
        in_specs=[
            pl.BlockSpec((NUM_EXPERTS, D_MODEL), lambda i: (0, 0)),
            pl.BlockSpec((TOK_BLK, D_MODEL), lambda i: (i, 0)),
        ],
        out_specs=pl.BlockSpec((NUM_EXPERTS, TOK_BLK), lambda i: (0, i)),
        out_shape=jax.ShapeDtypeStruct((NUM_EXPERTS, TOKENS), jnp.float32),
    )(W.astype(jnp.float32), x.astype(jnp.float32))

    weights, indices = pl.kernel(
        _routing_body,
        out_type=[
            jax.ShapeDtypeStruct((TOKENS,), jnp.float32),
            jax.ShapeDtypeStruct((TOKENS,), jnp.int32),
        ],
        mesh=plsc.VectorSubcoreMesh(core_axis_name="c", subcore_axis_name="s"),
        scratch_types=[
            pltpu.VMEM((NUM_EXPERTS, TPW), jnp.float32),
            pltpu.VMEM((TPW,), jnp.float32),
            pltpu.VMEM((TPW,), jnp.int32),
        ],
    )(logits_t)

    return (weights.reshape(TOKENS, 1).astype(x.dtype),
            indices.reshape(TOKENS, 1))
